# baseline (speedup 1.0000x reference)
import functools

import jax
import jax.numpy as jnp
from jax import lax
from jax.experimental import pallas as pl
from jax.experimental.pallas import tpu as pltpu
from jax.experimental.pallas import tpu_sc as plsc

NN = 10000
NPAD = 10240
NC = 2
NS = 16
NW = NC * NS
ZN = NPAD // NS
F16 = 16
PR = NPAD // 8
NR = NN // 8

CHUNK = 2000
NBUF = 3

_f32 = jnp.float32
_i32 = jnp.int32


def _mesh():
    return plsc.VectorSubcoreMesh(core_axis_name="c", subcore_axis_name="s")


def _make_deg_kernel(E):
    epw = E // NW
    nchunk = epw // CHUNK

    @functools.partial(
        pl.kernel,
        out_type=jax.ShapeDtypeStruct((NC * NPAD,), _f32),
        mesh=_mesh(),
        scratch_types=[
            pltpu.VMEM((CHUNK,), _i32),
            pltpu.VMEM((CHUNK,), _f32),
            pltpu.VMEM_SHARED((NPAD,), _f32),
            pltpu.VMEM((ZN,), _f32),
        ],
    )
    def deg_kernel(ei_hbm, ew_hbm, out_hbm, colv, ewv, deg_sh, zb):
        c = lax.axis_index("c")
        s = lax.axis_index("s")
        zero = jnp.zeros((16,), _f32)
        for j in range(ZN // 16):
            zb[pl.ds(j * 16, 16)] = zero
        pltpu.sync_copy(zb, deg_sh.at[pl.ds(s * ZN, ZN)])
        plsc.subcore_barrier()
        base = (c * NS + s) * epw
        for k in range(nchunk):
            off = base + k * CHUNK
            pltpu.sync_copy(ei_hbm.at[pl.ds(E + off, CHUNK)], colv)
            pltpu.sync_copy(ew_hbm.at[pl.ds(off, CHUNK)], ewv)
            pltpu.sync_copy(ewv, deg_sh.at[colv], add=True)
        plsc.subcore_barrier()
        pltpu.sync_copy(deg_sh.at[pl.ds(s * ZN, ZN)],
                        out_hbm.at[pl.ds(c * NPAD + s * ZN, ZN)])

    return deg_kernel


def _make_agg_kernel(E):
    epw = E // NW
    nchunk = epw // CHUNK

    scratch = (
        [pltpu.VMEM((CHUNK,), _i32)] * NBUF +
        [pltpu.VMEM((CHUNK,), _i32)] * NBUF +
        [pltpu.VMEM((CHUNK,), _f32)] * NBUF +
        [pltpu.VMEM((CHUNK,), _f32)] * NBUF +
        [pltpu.VMEM((CHUNK, F16), _f32)] * 2 +
        [pltpu.VMEM_SHARED((NPAD, F16), _f32),
         pltpu.VMEM((64, F16), _f32)] +
        [pltpu.SemaphoreType.DMA] * (4 * NBUF)
    )

    @functools.partial(
        pl.kernel,
        out_type=jax.ShapeDtypeStruct((NC, NPAD, F16), _f32),
        mesh=_mesh(),
        scratch_types=scratch,
        compiler_params=pltpu.CompilerParams(use_tc_tiling_on_sc=False),
    )
    def agg_kernel(hs_hbm, ei_hbm, ew_hbm, dinv_hbm, out_hbm, *bufs):
        rowv = bufs[0:NBUF]
        colv = bufs[NBUF:2 * NBUF]
        ewv = bufs[2 * NBUF:3 * NBUF]
        drv = bufs[3 * NBUF:4 * NBUF]
        msg = bufs[4 * NBUF:4 * NBUF + 2]
        acc_sh, zb = bufs[4 * NBUF + 2:4 * NBUF + 4]
        sems = bufs[4 * NBUF + 4:]
        sem_cp = sems[0:NBUF]
        sem_g = sems[NBUF:2 * NBUF]
        sem_d = sems[2 * NBUF:3 * NBUF]
        sem_s = sems[3 * NBUF:4 * NBUF]

        c = lax.axis_index("c")
        s = lax.axis_index("s")
        zero = jnp.zeros((16,), _f32)

        def zbody(i, carry):
            zb[i, :] = zero
            return carry

        lax.fori_loop(0, 64, zbody, 0)
        for j in range(ZN // 64):
            pltpu.sync_copy(zb, acc_sh.at[pl.ds(s * ZN + j * 64, 64)])
        plsc.subcore_barrier()

        base = (c * NS + s) * epw

        def issue_copies(k):
            b = k % NBUF
            off = base + k * CHUNK
            r = pltpu.async_copy(ei_hbm.at[pl.ds(off, CHUNK)], rowv[b],
                                 sem_cp[b])
            cc = pltpu.async_copy(ei_hbm.at[pl.ds(E + off, CHUNK)], colv[b],
                                  sem_cp[b])
            w = pltpu.async_copy(ew_hbm.at[pl.ds(off, CHUNK)], ewv[b],
                                 sem_cp[b])
            return r, cc, w

        def issue_gather(k, cps):
            for d in cps:
                d.wait()
            b = k % NBUF
            g = pltpu.async_copy(hs_hbm.at[rowv[b]], msg[k % 2], sem_g[k % 2])
            dg = pltpu.async_copy(dinv_hbm.at[rowv[b]], drv[b], sem_d[b])
            return g, dg

        def scale(k):
            b = k % NBUF
            mb = msg[k % 2]
            wb = ewv[b]
            db_ = drv[b]

            def sbody(i, carry):
                w16 = wb[pl.ds(i * 16, 16)] * db_[pl.ds(i * 16, 16)]
                for u in range(16):
                    j = i * 16 + u
                    mb[j, :] = mb[j, :] * w16[u]
                return carry

            lax.fori_loop(0, CHUNK // 16, sbody, 0)

        def issue_scatter(k):
            b = k % NBUF
            return pltpu.async_copy(msg[k % 2], acc_sh.at[colv[b]],
                                    sem_s[k % 2], add=True)

        cps = [None] * nchunk
        gth = [None] * nchunk
        sct = [None] * nchunk
        cps[0] = issue_copies(0)
        if nchunk > 1:
            cps[1] = issue_copies(1)
        gth[0] = issue_gather(0, cps[0])
        for k in range(nchunk):
            for g in gth[k]:
                g.wait()
            if k + 1 < nchunk:
                if k >= 1:
                    sct[k - 1].wait()
                gth[k + 1] = issue_gather(k + 1, cps[k + 1])
            if k + 2 < nchunk:
                cps[k + 2] = issue_copies(k + 2)
            scale(k)
            sct[k] = issue_scatter(k)
        for k in range(max(0, nchunk - 2), nchunk):
            if sct[k] is not None and k >= nchunk - 2:
                sct[k].wait()
        plsc.subcore_barrier()

        ob = msg[0].at[pl.ds(0, ZN), :]
        hb = msg[1].at[pl.ds(0, ZN), :]
        db = ewv[0].at[pl.ds(0, ZN)]
        pltpu.sync_copy(acc_sh.at[pl.ds(s * ZN, ZN)], ob)
        pltpu.sync_copy(dinv_hbm.at[pl.ds(s * ZN, ZN)], db)
        pltpu.sync_copy(hs_hbm.at[pl.ds(s * ZN, ZN)], hb)

        @pl.when(c == 0)
        def _():
            def obody0(i, carry):
                d16 = db[pl.ds(i * 16, 16)]
                for u in range(16):
                    j = i * 16 + u
                    dv = d16[u]
                    ob[j, :] = dv * (ob[j, :] + dv * hb[j, :])
                return carry

            lax.fori_loop(0, ZN // 16, obody0, 0)

        @pl.when(c != 0)
        def _():
            def obody1(i, carry):
                d16 = db[pl.ds(i * 16, 16)]
                for u in range(16):
                    j = i * 16 + u
                    ob[j, :] = ob[j, :] * d16[u]
                return carry

            lax.fori_loop(0, ZN // 16, obody1, 0)

        pltpu.sync_copy(ob, out_hbm.at[c, pl.ds(s * ZN, ZN)])

    return agg_kernel


def _t_dinv(degf_ref, o_ref):
    half = (NC * NPAD // 128) // 2
    deg = degf_ref[:half, :] + degf_ref[half:, :] + 1.0
    o_ref[:] = lax.rsqrt(deg)


def _t_lin1(xp_ref, w_ref, o_ref):
    o_ref[:NR, :] = jnp.dot(xp_ref[:], w_ref[:], preferred_element_type=_f32)
    o_ref[NR:, :] = jnp.zeros((PR - NR, 128), _f32)


def _t_mid(sp_ref, b_ref, w_ref, o_ref):
    z = jnp.maximum(sp_ref[0] + sp_ref[1] + b_ref[:], 0.0)
    o_ref[:] = jnp.dot(z, w_ref[:], preferred_element_type=_f32)


def _t_final(sp_ref, b_ref, o_ref):
    z = sp_ref[0, :NR, :] + sp_ref[1, :NR, :] + b_ref[:]
    w = z
    for sh in (8, 4, 2, 1):
        w = jnp.maximum(w, pltpu.roll(w, 128 - sh, axis=1))
    a = lax.broadcasted_iota(_i32, (128, 128), 0)
    b = lax.broadcasted_iota(_i32, (128, 128), 1)
    msel = (a == (b // F16) * F16).astype(_f32)
    gsum = ((a // F16) == (b // F16)).astype(_f32)
    m = jnp.dot(w, msel, preferred_element_type=_f32)
    e = jnp.exp(z - m)
    s = jnp.dot(e, gsum, preferred_element_type=_f32)
    o_ref[:] = z - m - jnp.log(s)


def kernel(x, edge_index, edge_weight, W1, b1, W2, b2):
    E = edge_index.shape[1]

    deg_kernel = _make_deg_kernel(E)
    agg_kernel = _make_agg_kernel(E)

    ei_flat = edge_index.reshape(2 * E)
    eye8 = jnp.eye(8, dtype=_f32)
    W1big = jnp.kron(eye8, W1)
    W2big = jnp.kron(eye8, W2)
    b1p = jnp.tile(b1, 8).reshape(1, 128)
    b2p = jnp.tile(b2, 8).reshape(1, 128)

    degf = deg_kernel(ei_flat, edge_weight)

    dinv80 = pl.pallas_call(
        _t_dinv,
        out_shape=jax.ShapeDtypeStruct((NPAD // 128, 128), _f32),
    )(degf.reshape(NC * NPAD // 128, 128))
    dinv_flat = dinv80.reshape(NPAD)

    h1p = pl.pallas_call(
        _t_lin1,
        out_shape=jax.ShapeDtypeStruct((PR, 128), _f32),
    )(x.reshape(NR, 1024), W1big)

    s1 = agg_kernel(h1p.reshape(NPAD, F16), ei_flat, edge_weight, dinv_flat)

    h2p = pl.pallas_call(
        _t_mid,
        out_shape=jax.ShapeDtypeStruct((PR, 128), _f32),
    )(s1.reshape(NC, PR, 128), b1p, W2big)

    s2 = agg_kernel(h2p.reshape(NPAD, F16), ei_flat, edge_weight, dinv_flat)

    outp = pl.pallas_call(
        _t_final,
        out_shape=jax.ShapeDtypeStruct((NR, 128), _f32),
    )(s2.reshape(NC, PR, 128), b2p)
    return outp.reshape(NN, F16)


# speedup vs baseline: 1.4838x; 1.4838x over previous
import functools

import jax
import jax.numpy as jnp
from jax import lax
from jax.experimental import pallas as pl
from jax.experimental.pallas import tpu as pltpu
from jax.experimental.pallas import tpu_sc as plsc

NN = 10000
NPAD = 10240
NC = 2
NS = 16
NW = NC * NS
ZN = NPAD // NS
F16 = 16
PR = NPAD // 8
NR = NN // 8

CHUNK = 2000
NBUF = 3

_f32 = jnp.float32
_i32 = jnp.int32


def _mesh():
    return plsc.VectorSubcoreMesh(core_axis_name="c", subcore_axis_name="s")


def _make_deg_kernel(E):
    epw = E // NW
    nchunk = epw // CHUNK

    @functools.partial(
        pl.kernel,
        out_type=jax.ShapeDtypeStruct((NC * NPAD,), _f32),
        mesh=_mesh(),
        scratch_types=[
            pltpu.VMEM((CHUNK,), _i32),
            pltpu.VMEM((CHUNK,), _f32),
            pltpu.VMEM_SHARED((NPAD,), _f32),
            pltpu.VMEM((ZN,), _f32),
        ],
    )
    def deg_kernel(ei_hbm, ew_hbm, out_hbm, colv, ewv, deg_sh, zb):
        c = lax.axis_index("c")
        s = lax.axis_index("s")
        zero = jnp.zeros((16,), _f32)
        for j in range(ZN // 16):
            zb[pl.ds(j * 16, 16)] = zero
        pltpu.sync_copy(zb, deg_sh.at[pl.ds(s * ZN, ZN)])
        plsc.subcore_barrier()
        base = (c * NS + s) * epw
        for k in range(nchunk):
            off = base + k * CHUNK
            pltpu.sync_copy(ei_hbm.at[pl.ds(E + off, CHUNK)], colv)
            pltpu.sync_copy(ew_hbm.at[pl.ds(off, CHUNK)], ewv)
            pltpu.sync_copy(ewv, deg_sh.at[colv], add=True)
        plsc.subcore_barrier()
        pltpu.sync_copy(deg_sh.at[pl.ds(s * ZN, ZN)],
                        out_hbm.at[pl.ds(c * NPAD + s * ZN, ZN)])

    return deg_kernel


def _make_agg_kernel(E):
    epw = E // NW
    nchunk = epw // CHUNK

    scratch = (
        [pltpu.VMEM((CHUNK,), _i32)] * NBUF +
        [pltpu.VMEM((CHUNK,), _i32)] * NBUF +
        [pltpu.VMEM((CHUNK,), _f32)] * NBUF +
        [pltpu.VMEM((CHUNK, F16), _f32)] * 2 +
        [pltpu.VMEM_SHARED((NPAD, F16), _f32),
         pltpu.VMEM_SHARED((NPAD, F16), _f32),
         pltpu.VMEM((64, F16), _f32)] +
        [pltpu.SemaphoreType.DMA] * (3 * NBUF)
    )

    @functools.partial(
        pl.kernel,
        out_type=jax.ShapeDtypeStruct((NC, NPAD, F16), _f32),
        mesh=_mesh(),
        scratch_types=scratch,
        compiler_params=pltpu.CompilerParams(use_tc_tiling_on_sc=False),
    )
    def agg_kernel(hs_hbm, ei_hbm, ew_hbm, dinv_hbm, out_hbm, *bufs):
        rowv = bufs[0:NBUF]
        colv = bufs[NBUF:2 * NBUF]
        ewv = bufs[2 * NBUF:3 * NBUF]
        msg = bufs[3 * NBUF:3 * NBUF + 2]
        tab_sh, acc_sh, zb = bufs[3 * NBUF + 2:3 * NBUF + 5]
        sems = bufs[3 * NBUF + 5:]
        sem_cp = sems[0:NBUF]
        sem_g = sems[NBUF:2 * NBUF]
        sem_s = sems[2 * NBUF:3 * NBUF]

        c = lax.axis_index("c")
        s = lax.axis_index("s")
        zero = jnp.zeros((16,), _f32)

        def zbody(i, carry):
            zb[i, :] = zero
            return carry

        lax.fori_loop(0, 64, zbody, 0)
        for j in range(ZN // 64):
            pltpu.sync_copy(zb, acc_sh.at[pl.ds(s * ZN + j * 64, 64)])

        pb = msg[0].at[pl.ds(0, ZN), :]
        db0 = ewv[0].at[pl.ds(0, ZN)]
        pltpu.sync_copy(hs_hbm.at[pl.ds(s * ZN, ZN)], pb)
        pltpu.sync_copy(dinv_hbm.at[pl.ds(s * ZN, ZN)], db0)

        def pbody(i, carry):
            d16 = db0[pl.ds(i * 16, 16)]
            for u in range(16):
                j = i * 16 + u
                pb[j, :] = pb[j, :] * d16[u]
            return carry

        lax.fori_loop(0, ZN // 16, pbody, 0)
        pltpu.sync_copy(pb, tab_sh.at[pl.ds(s * ZN, ZN)])
        plsc.subcore_barrier()

        base = (c * NS + s) * epw

        def issue_copies(k):
            b = k % NBUF
            off = base + k * CHUNK
            r = pltpu.async_copy(ei_hbm.at[pl.ds(off, CHUNK)], rowv[b],
                                 sem_cp[b])
            cc = pltpu.async_copy(ei_hbm.at[pl.ds(E + off, CHUNK)], colv[b],
                                  sem_cp[b])
            w = pltpu.async_copy(ew_hbm.at[pl.ds(off, CHUNK)], ewv[b],
                                 sem_cp[b])
            return r, cc, w

        def issue_gather(k, cps):
            for d in cps:
                d.wait()
            b = k % NBUF
            return pltpu.async_copy(tab_sh.at[rowv[b]], msg[k % 2],
                                    sem_g[k % 2])

        def scale(k):
            b = k % NBUF
            mb = msg[k % 2]
            wb = ewv[b]

            def sbody(i, carry):
                w16 = wb[pl.ds(i * 16, 16)]
                for u in range(16):
                    j = i * 16 + u
                    mb[j, :] = mb[j, :] * w16[u]
                return carry

            lax.fori_loop(0, CHUNK // 16, sbody, 0)

        def issue_scatter(k):
            b = k % NBUF
            return pltpu.async_copy(msg[k % 2], acc_sh.at[colv[b]],
                                    sem_s[k % 2], add=True)

        cps = [None] * nchunk
        gth = [None] * nchunk
        sct = [None] * nchunk
        cps[0] = issue_copies(0)
        if nchunk > 1:
            cps[1] = issue_copies(1)
        gth[0] = issue_gather(0, cps[0])
        for k in range(nchunk):
            gth[k].wait()
            if k + 1 < nchunk:
                if k >= 1:
                    sct[k - 1].wait()
                gth[k + 1] = issue_gather(k + 1, cps[k + 1])
            if k + 2 < nchunk:
                cps[k + 2] = issue_copies(k + 2)
            scale(k)
            sct[k] = issue_scatter(k)
        for k in range(max(0, nchunk - 2), nchunk):
            if sct[k] is not None and k >= nchunk - 2:
                sct[k].wait()
        plsc.subcore_barrier()

        ob = msg[0].at[pl.ds(0, ZN), :]
        hb = msg[1].at[pl.ds(0, ZN), :]
        db = ewv[0].at[pl.ds(0, ZN)]
        pltpu.sync_copy(acc_sh.at[pl.ds(s * ZN, ZN)], ob)
        pltpu.sync_copy(dinv_hbm.at[pl.ds(s * ZN, ZN)], db)
        pltpu.sync_copy(tab_sh.at[pl.ds(s * ZN, ZN)], hb)

        @pl.when(c == 0)
        def _():
            def obody0(i, carry):
                d16 = db[pl.ds(i * 16, 16)]
                for u in range(16):
                    j = i * 16 + u
                    ob[j, :] = d16[u] * (ob[j, :] + hb[j, :])
                return carry

            lax.fori_loop(0, ZN // 16, obody0, 0)

        @pl.when(c != 0)
        def _():
            def obody1(i, carry):
                d16 = db[pl.ds(i * 16, 16)]
                for u in range(16):
                    j = i * 16 + u
                    ob[j, :] = ob[j, :] * d16[u]
                return carry

            lax.fori_loop(0, ZN // 16, obody1, 0)

        pltpu.sync_copy(ob, out_hbm.at[c, pl.ds(s * ZN, ZN)])

    return agg_kernel


def _t_dinv(degf_ref, o_ref):
    half = (NC * NPAD // 128) // 2
    deg = degf_ref[:half, :] + degf_ref[half:, :] + 1.0
    o_ref[:] = lax.rsqrt(deg)


def _t_lin1(xp_ref, w_ref, o_ref):
    o_ref[:NR, :] = jnp.dot(xp_ref[:], w_ref[:], preferred_element_type=_f32)
    o_ref[NR:, :] = jnp.zeros((PR - NR, 128), _f32)


def _t_mid(sp_ref, b_ref, w_ref, o_ref):
    z = jnp.maximum(sp_ref[0] + sp_ref[1] + b_ref[:], 0.0)
    o_ref[:] = jnp.dot(z, w_ref[:], preferred_element_type=_f32)


def _t_final(sp_ref, b_ref, o_ref):
    z = sp_ref[0, :NR, :] + sp_ref[1, :NR, :] + b_ref[:]
    w = z
    for sh in (8, 4, 2, 1):
        w = jnp.maximum(w, pltpu.roll(w, 128 - sh, axis=1))
    a = lax.broadcasted_iota(_i32, (128, 128), 0)
    b = lax.broadcasted_iota(_i32, (128, 128), 1)
    msel = (a == (b // F16) * F16).astype(_f32)
    gsum = ((a // F16) == (b // F16)).astype(_f32)
    m = jnp.dot(w, msel, preferred_element_type=_f32)
    e = jnp.exp(z - m)
    s = jnp.dot(e, gsum, preferred_element_type=_f32)
    o_ref[:] = z - m - jnp.log(s)


def kernel(x, edge_index, edge_weight, W1, b1, W2, b2):
    E = edge_index.shape[1]

    deg_kernel = _make_deg_kernel(E)
    agg_kernel = _make_agg_kernel(E)

    ei_flat = edge_index.reshape(2 * E)
    eye8 = jnp.eye(8, dtype=_f32)
    W1big = jnp.kron(eye8, W1)
    W2big = jnp.kron(eye8, W2)
    b1p = jnp.tile(b1, 8).reshape(1, 128)
    b2p = jnp.tile(b2, 8).reshape(1, 128)

    degf = deg_kernel(ei_flat, edge_weight)

    dinv80 = pl.pallas_call(
        _t_dinv,
        out_shape=jax.ShapeDtypeStruct((NPAD // 128, 128), _f32),
    )(degf.reshape(NC * NPAD // 128, 128))
    dinv_flat = dinv80.reshape(NPAD)

    h1p = pl.pallas_call(
        _t_lin1,
        out_shape=jax.ShapeDtypeStruct((PR, 128), _f32),
    )(x.reshape(NR, 1024), W1big)

    s1 = agg_kernel(h1p.reshape(NPAD, F16), ei_flat, edge_weight, dinv_flat)

    h2p = pl.pallas_call(
        _t_mid,
        out_shape=jax.ShapeDtypeStruct((PR, 128), _f32),
    )(s1.reshape(NC, PR, 128), b1p, W2big)

    s2 = agg_kernel(h2p.reshape(NPAD, F16), ei_flat, edge_weight, dinv_flat)

    outp = pl.pallas_call(
        _t_final,
        out_shape=jax.ShapeDtypeStruct((NR, 128), _f32),
    )(s2.reshape(NC, PR, 128), b2p)
    return outp.reshape(NN, F16)
